# baseline (speedup 1.0000x reference)
import jax
import jax.numpy as jnp
from jax import lax
from jax.experimental import pallas as pl
from jax.experimental.pallas import tpu as pltpu
from jax.experimental.pallas import tpu_sc as plsc

N_VOCAB = 1000000
K = 16
BATCH = 16384
FIELDS = 26

NC = 2
NS = 16
NW = NC * NS
SAMPLES_PER_W = BATCH // NW
CHUNK = 128
N_CHUNKS = SAMPLES_PER_W // CHUNK
IDX_ROWS = CHUNK * FIELDS // 128

N_FULL_TILES = N_VOCAB // 128
TAIL = N_VOCAB - N_FULL_TILES * 128


def _transpose_body(embt_hbm, tail_hbm, out_hbm, in_v, out_v, sem):
    wid = lax.axis_index("s") * NC + lax.axis_index("c")
    iota = lax.iota(jnp.int32, 16)

    n_mine = (N_FULL_TILES - wid + NW - 1) // NW

    def tile_body(i, _):
        vt = wid + i * NW
        col_off = pl.multiple_of(vt * 128, 128)
        pltpu.async_copy(
            embt_hbm.at[:, pl.ds(col_off, 128)], in_v, sem).wait()

        def col_body(v, _):
            vec = plsc.load_gather(in_v, [iota, jnp.full((16,), v, jnp.int32)])
            out_v[pl.ds(v * K, K)] = vec
            return 0

        lax.fori_loop(0, 128, col_body, 0)
        pltpu.sync_copy(out_v, out_hbm.at[pl.ds(col_off * K, 128 * K)])
        return 0

    lax.fori_loop(0, n_mine, tile_body, 0)

    @pl.when(wid == NW - 1)
    def _():
        pltpu.sync_copy(tail_hbm, out_hbm.at[pl.ds(N_FULL_TILES * 128 * K,
                                                   TAIL * K)])


def _fm_body(x_hbm, emb_hbm, bias_hbm, w0_hbm, out_hbm,
             idx_v, emb_v, bias_v, out_v, w0_v, esem, bsem):
    wid = lax.axis_index("s") * NC + lax.axis_index("c")

    pltpu.sync_copy(w0_hbm, w0_v)
    w0vec = w0_v[...]

    iota = lax.iota(jnp.int32, 16)
    iota26 = iota * FIELDS
    zeros16 = jnp.zeros((16,), jnp.int32)

    for c in range(N_CHUNKS):
        idx_off = (wid * N_CHUNKS + c) * (CHUNK * FIELDS)
        pltpu.sync_copy(x_hbm.at[pl.ds(idx_off, CHUNK * FIELDS)], idx_v)

        descs = []
        for j in range(IDX_ROWS):
            descs.append(pltpu.async_copy(
                emb_hbm.at[idx_v.at[pl.ds(j * 128, 128)]],
                emb_v.at[pl.ds(j * 128, 128)], esem))
            descs.append(pltpu.async_copy(
                bias_hbm.at[idx_v.at[pl.ds(j * 128, 128)]],
                bias_v.at[pl.ds(j * 128, 128)], bsem))
        for d in descs:
            d.wait()

        def group_body(g, _):
            rbase = iota26 + g * (16 * FIELDS)

            def k_body(k, acc):
                cols = jnp.full((16,), k, jnp.int32)
                s = jnp.zeros((16,), jnp.float32)
                q = jnp.zeros((16,), jnp.float32)
                for f in range(FIELDS):
                    v = plsc.load_gather(emb_v, [rbase + f, cols])
                    s = s + v
                    q = q + v * v
                return acc + (s * s - q)

            pair = lax.fori_loop(0, K, k_body, jnp.zeros((16,), jnp.float32))
            bacc = jnp.zeros((16,), jnp.float32)
            for f in range(FIELDS):
                bacc = bacc + plsc.load_gather(bias_v, [rbase + f])
            t = w0vec + bacc + 0.5 * pair
            out_v[pl.ds(g * 16, 16)] = 5.5 / (1.0 + jnp.exp(-t))
            return 0

        lax.fori_loop(0, CHUNK // 16, group_body, 0)

        out_off = wid * SAMPLES_PER_W + c * CHUNK
        pltpu.sync_copy(out_v, out_hbm.at[pl.ds(out_off, CHUNK)])


def _sc_mesh():
    return plsc.VectorSubcoreMesh(core_axis_name="c", subcore_axis_name="s")


@jax.jit
def _fm_call(X, emb, bias, w0):
    xflat = X.reshape(BATCH * FIELDS)
    w0b = jnp.broadcast_to(w0, (16,))
    tail = emb[N_FULL_TILES * 128:, :].reshape(TAIL * K)

    table = pl.kernel(
        _transpose_body,
        out_type=jax.ShapeDtypeStruct((N_VOCAB * K,), jnp.float32),
        mesh=_sc_mesh(),
        scratch_types=[
            pltpu.VMEM((16, 128), jnp.float32),
            pltpu.VMEM((128 * K,), jnp.float32),
            pltpu.SemaphoreType.DMA,
        ],
        compiler_params=pltpu.CompilerParams(
            needs_layout_passes=False, use_tc_tiling_on_sc=True),
    )(emb.T, tail)

    return pl.kernel(
        _fm_body,
        out_type=jax.ShapeDtypeStruct((BATCH,), jnp.float32),
        mesh=_sc_mesh(),
        scratch_types=[
            pltpu.VMEM((CHUNK * FIELDS,), jnp.int32),
            pltpu.VMEM((CHUNK * FIELDS, K), jnp.float32),
            pltpu.VMEM((CHUNK * FIELDS,), jnp.float32),
            pltpu.VMEM((CHUNK,), jnp.float32),
            pltpu.VMEM((16,), jnp.float32),
            pltpu.SemaphoreType.DMA,
            pltpu.SemaphoreType.DMA,
        ],
        compiler_params=pltpu.CompilerParams(
            needs_layout_passes=False, use_tc_tiling_on_sc=False),
    )(xflat, table.reshape(N_VOCAB, K), bias.T.reshape(N_VOCAB), w0b)


def kernel(X, embeddings, bias, w0):
    return _fm_call(X.astype(jnp.int32), embeddings,
                    bias.astype(jnp.float32), w0.astype(jnp.float32))


# speedup vs baseline: 1.3671x; 1.3671x over previous
import jax
import jax.numpy as jnp
from jax import lax
from jax.experimental import pallas as pl
from jax.experimental.pallas import tpu as pltpu
from jax.experimental.pallas import tpu_sc as plsc

N_VOCAB = 1000000
K = 16
BATCH = 16384
FIELDS = 26

NC = 2
NS = 16
NW = NC * NS
SAMPLES_PER_W = BATCH // NW
CHUNK = 128
N_CHUNKS = SAMPLES_PER_W // CHUNK
IDX_ROWS = CHUNK * FIELDS // 128

TCOLS = 512
N_TCHUNKS = N_VOCAB // TCOLS
TAIL = N_VOCAB - N_TCHUNKS * TCOLS


def _transpose_body(embt_hbm, tail_hbm, out_hbm,
                    in0, in1, ou0, ou1, is0, is1, os0, os1):
    wid = lax.axis_index("s") * NC + lax.axis_index("c")
    iota = lax.iota(jnp.int32, 16)
    ins, outs, iss, oss = (in0, in1), (ou0, ou1), (is0, is1), (os0, os1)

    n_mine = (N_TCHUNKS - wid + NW - 1) // NW

    def col_off(j):
        return pl.multiple_of((wid + j * NW) * TCOLS, TCOLS)

    for b in range(2):
        pltpu.async_copy(embt_hbm.at[:, pl.ds(col_off(b), TCOLS)],
                         ins[b], iss[b])

    def half_body(i, _):
        for b in range(2):
            j = 2 * i + b

            @pl.when(j < n_mine)
            def _(b=b, j=j):
                co = col_off(j)
                pltpu.make_async_copy(
                    embt_hbm.at[:, pl.ds(co, TCOLS)], ins[b], iss[b]).wait()

                @pl.when(j >= 2)
                def _():
                    pltpu.make_async_copy(
                        outs[b], out_hbm.at[pl.ds(0, TCOLS * K)],
                        oss[b]).wait()

                def grp(g, _):
                    base = g * 16
                    for t in range(16):
                        v = base + t
                        vec = plsc.load_gather(
                            ins[b], [iota, jnp.full((16,), v, jnp.int32)])
                        outs[b][pl.ds(v * K, K)] = vec
                    return 0

                lax.fori_loop(0, TCOLS // 16, grp, 0)
                pltpu.async_copy(outs[b],
                                 out_hbm.at[pl.ds(co * K, TCOLS * K)], oss[b])

                @pl.when(j + 2 < n_mine)
                def _():
                    co2 = col_off(j + 2)
                    pltpu.async_copy(embt_hbm.at[:, pl.ds(co2, TCOLS)],
                                     ins[b], iss[b])
        return 0

    lax.fori_loop(0, (N_TCHUNKS // NW + 2) // 2, half_body, 0)

    for b in range(2):
        pltpu.make_async_copy(
            outs[b], out_hbm.at[pl.ds(0, TCOLS * K)], oss[b]).wait()

    @pl.when(wid == NW - 1)
    def _():
        pltpu.sync_copy(tail_hbm, out_hbm.at[pl.ds(N_TCHUNKS * TCOLS * K,
                                                   TAIL * K)])


def _fm_body(x_hbm, emb_hbm, bias_hbm, w0_hbm, out_hbm,
             idx_v, emb_v, bias_v, out_v, w0_v, esem, bsem):
    wid = lax.axis_index("s") * NC + lax.axis_index("c")

    pltpu.sync_copy(w0_hbm, w0_v)
    w0vec = w0_v[...]

    iota = lax.iota(jnp.int32, 16)
    iota26 = iota * FIELDS
    zeros16 = jnp.zeros((16,), jnp.int32)

    for c in range(N_CHUNKS):
        idx_off = (wid * N_CHUNKS + c) * (CHUNK * FIELDS)
        pltpu.sync_copy(x_hbm.at[pl.ds(idx_off, CHUNK * FIELDS)], idx_v)

        descs = []
        for j in range(IDX_ROWS):
            descs.append(pltpu.async_copy(
                emb_hbm.at[idx_v.at[pl.ds(j * 128, 128)]],
                emb_v.at[pl.ds(j * 128, 128)], esem))
            descs.append(pltpu.async_copy(
                bias_hbm.at[idx_v.at[pl.ds(j * 128, 128)]],
                bias_v.at[pl.ds(j * 128, 128)], bsem))
        for d in descs:
            d.wait()

        def group_body(g, _):
            rbase = iota26 + g * (16 * FIELDS)

            def k_body(k, acc):
                cols = jnp.full((16,), k, jnp.int32)
                s = jnp.zeros((16,), jnp.float32)
                q = jnp.zeros((16,), jnp.float32)
                for f in range(FIELDS):
                    v = plsc.load_gather(emb_v, [rbase + f, cols])
                    s = s + v
                    q = q + v * v
                return acc + (s * s - q)

            pair = lax.fori_loop(0, K, k_body, jnp.zeros((16,), jnp.float32))
            bacc = jnp.zeros((16,), jnp.float32)
            for f in range(FIELDS):
                bacc = bacc + plsc.load_gather(bias_v, [rbase + f])
            t = w0vec + bacc + 0.5 * pair
            out_v[pl.ds(g * 16, 16)] = 5.5 / (1.0 + jnp.exp(-t))
            return 0

        lax.fori_loop(0, CHUNK // 16, group_body, 0)

        out_off = wid * SAMPLES_PER_W + c * CHUNK
        pltpu.sync_copy(out_v, out_hbm.at[pl.ds(out_off, CHUNK)])


def _sc_mesh():
    return plsc.VectorSubcoreMesh(core_axis_name="c", subcore_axis_name="s")


@jax.jit
def _fm_call(X, emb, bias, w0):
    xflat = X.reshape(BATCH * FIELDS)
    w0b = jnp.broadcast_to(w0, (16,))
    tail = emb[N_TCHUNKS * TCOLS:, :].reshape(TAIL * K)

    table = pl.kernel(
        _transpose_body,
        out_type=jax.ShapeDtypeStruct((N_VOCAB * K,), jnp.float32),
        mesh=_sc_mesh(),
        scratch_types=[
            pltpu.VMEM((16, TCOLS), jnp.float32),
            pltpu.VMEM((16, TCOLS), jnp.float32),
            pltpu.VMEM((TCOLS * K,), jnp.float32),
            pltpu.VMEM((TCOLS * K,), jnp.float32),
            pltpu.SemaphoreType.DMA,
            pltpu.SemaphoreType.DMA,
            pltpu.SemaphoreType.DMA,
            pltpu.SemaphoreType.DMA,
        ],
        compiler_params=pltpu.CompilerParams(
            needs_layout_passes=False, use_tc_tiling_on_sc=True),
    )(emb.T, tail)

    return pl.kernel(
        _fm_body,
        out_type=jax.ShapeDtypeStruct((BATCH,), jnp.float32),
        mesh=_sc_mesh(),
        scratch_types=[
            pltpu.VMEM((CHUNK * FIELDS,), jnp.int32),
            pltpu.VMEM((CHUNK * FIELDS, K), jnp.float32),
            pltpu.VMEM((CHUNK * FIELDS,), jnp.float32),
            pltpu.VMEM((CHUNK,), jnp.float32),
            pltpu.VMEM((16,), jnp.float32),
            pltpu.SemaphoreType.DMA,
            pltpu.SemaphoreType.DMA,
        ],
        compiler_params=pltpu.CompilerParams(
            needs_layout_passes=False, use_tc_tiling_on_sc=False),
    )(xflat, table.reshape(N_VOCAB, K), bias.T.reshape(N_VOCAB), w0b)


def kernel(X, embeddings, bias, w0):
    return _fm_call(X.astype(jnp.int32), embeddings,
                    bias.astype(jnp.float32), w0.astype(jnp.float32))


# speedup vs baseline: 2.7956x; 2.0449x over previous
import jax
import jax.numpy as jnp
from jax import lax
from jax.experimental import pallas as pl
from jax.experimental.pallas import tpu as pltpu
from jax.experimental.pallas import tpu_sc as plsc

N_VOCAB = 1000000
K = 16
BATCH = 16384
FIELDS = 26

NC = 2
NS = 16
NW = NC * NS
SAMPLES_PER_W = BATCH // NW
CHUNK = 128
N_CHUNKS = SAMPLES_PER_W // CHUNK
IDX_ROWS = CHUNK * FIELDS // 128

TCOLS = 512
N_TCHUNKS = N_VOCAB // TCOLS
TAIL = N_VOCAB - N_TCHUNKS * TCOLS


def _transpose_body(embt_hbm, tail_hbm, out_hbm,
                    in0, in1, ou0, ou1, is0, is1, os0, os1):
    wid = lax.axis_index("s") * NC + lax.axis_index("c")
    iota16k = lax.iota(jnp.int32, 16) * K
    ins, outs, iss, oss = (in0, in1), (ou0, ou1), (is0, is1), (os0, os1)

    n_mine = (N_TCHUNKS - wid + NW - 1) // NW

    def col_off(j):
        return pl.multiple_of((wid + j * NW) * TCOLS, TCOLS)

    for b in range(2):
        pltpu.async_copy(embt_hbm.at[:, pl.ds(col_off(b), TCOLS)],
                         ins[b], iss[b])

    def half_body(i, _):
        for b in range(2):
            j = 2 * i + b

            @pl.when(j < n_mine)
            def _(b=b, j=j):
                co = col_off(j)
                pltpu.make_async_copy(
                    embt_hbm.at[:, pl.ds(co, TCOLS)], ins[b], iss[b]).wait()

                @pl.when(j >= 2)
                def _():
                    pltpu.make_async_copy(
                        outs[b], out_hbm.at[pl.ds(0, TCOLS * K)],
                        oss[b]).wait()

                def grp(g, _):
                    c = g * 16
                    cbase = c * K
                    for k in range(16):
                        vec = ins[b][k, pl.ds(c, 16)]
                        plsc.store_scatter(outs[b], [iota16k + (cbase + k)],
                                           vec)
                    return 0

                lax.fori_loop(0, TCOLS // 16, grp, 0)
                pltpu.async_copy(outs[b],
                                 out_hbm.at[pl.ds(co * K, TCOLS * K)], oss[b])

                @pl.when(j + 2 < n_mine)
                def _():
                    co2 = col_off(j + 2)
                    pltpu.async_copy(embt_hbm.at[:, pl.ds(co2, TCOLS)],
                                     ins[b], iss[b])
        return 0

    lax.fori_loop(0, (N_TCHUNKS // NW + 2) // 2, half_body, 0)

    for b in range(2):
        pltpu.make_async_copy(
            outs[b], out_hbm.at[pl.ds(0, TCOLS * K)], oss[b]).wait()

    @pl.when(wid == NW - 1)
    def _():
        pltpu.sync_copy(tail_hbm, out_hbm.at[pl.ds(N_TCHUNKS * TCOLS * K,
                                                   TAIL * K)])


def _fm_body(x_hbm, emb_hbm, bias_hbm, w0_hbm, out_hbm,
             idx_v, emb_v, bias_v, out_v, w0_v, esem, bsem):
    wid = lax.axis_index("s") * NC + lax.axis_index("c")

    pltpu.sync_copy(w0_hbm, w0_v)
    w0vec = w0_v[...]

    iota = lax.iota(jnp.int32, 16)
    iota26 = iota * FIELDS
    zeros16 = jnp.zeros((16,), jnp.int32)

    for c in range(N_CHUNKS):
        idx_off = (wid * N_CHUNKS + c) * (CHUNK * FIELDS)
        pltpu.sync_copy(x_hbm.at[pl.ds(idx_off, CHUNK * FIELDS)], idx_v)

        descs = []
        for j in range(IDX_ROWS):
            descs.append(pltpu.async_copy(
                emb_hbm.at[idx_v.at[pl.ds(j * 128, 128)]],
                emb_v.at[pl.ds(j * 128, 128)], esem))
            descs.append(pltpu.async_copy(
                bias_hbm.at[idx_v.at[pl.ds(j * 128, 128)]],
                bias_v.at[pl.ds(j * 128, 128)], bsem))
        for d in descs:
            d.wait()

        def group_body(g, _):
            rbase = iota26 + g * (16 * FIELDS)

            def k_body(k, acc):
                cols = jnp.full((16,), k, jnp.int32)
                s = jnp.zeros((16,), jnp.float32)
                q = jnp.zeros((16,), jnp.float32)
                for f in range(FIELDS):
                    v = plsc.load_gather(emb_v, [rbase + f, cols])
                    s = s + v
                    q = q + v * v
                return acc + (s * s - q)

            pair = lax.fori_loop(0, K, k_body, jnp.zeros((16,), jnp.float32))
            bacc = jnp.zeros((16,), jnp.float32)
            for f in range(FIELDS):
                bacc = bacc + plsc.load_gather(bias_v, [rbase + f])
            t = w0vec + bacc + 0.5 * pair
            out_v[pl.ds(g * 16, 16)] = 5.5 / (1.0 + jnp.exp(-t))
            return 0

        lax.fori_loop(0, CHUNK // 16, group_body, 0)

        out_off = wid * SAMPLES_PER_W + c * CHUNK
        pltpu.sync_copy(out_v, out_hbm.at[pl.ds(out_off, CHUNK)])


def _sc_mesh():
    return plsc.VectorSubcoreMesh(core_axis_name="c", subcore_axis_name="s")


@jax.jit
def _fm_call(X, emb, bias, w0):
    xflat = X.reshape(BATCH * FIELDS)
    w0b = jnp.broadcast_to(w0, (16,))
    tail = emb[N_TCHUNKS * TCOLS:, :].reshape(TAIL * K)

    table = pl.kernel(
        _transpose_body,
        out_type=jax.ShapeDtypeStruct((N_VOCAB * K,), jnp.float32),
        mesh=_sc_mesh(),
        scratch_types=[
            pltpu.VMEM((16, TCOLS), jnp.float32),
            pltpu.VMEM((16, TCOLS), jnp.float32),
            pltpu.VMEM((TCOLS * K,), jnp.float32),
            pltpu.VMEM((TCOLS * K,), jnp.float32),
            pltpu.SemaphoreType.DMA,
            pltpu.SemaphoreType.DMA,
            pltpu.SemaphoreType.DMA,
            pltpu.SemaphoreType.DMA,
        ],
        compiler_params=pltpu.CompilerParams(
            needs_layout_passes=False, use_tc_tiling_on_sc=True),
    )(emb.T, tail)

    return pl.kernel(
        _fm_body,
        out_type=jax.ShapeDtypeStruct((BATCH,), jnp.float32),
        mesh=_sc_mesh(),
        scratch_types=[
            pltpu.VMEM((CHUNK * FIELDS,), jnp.int32),
            pltpu.VMEM((CHUNK * FIELDS, K), jnp.float32),
            pltpu.VMEM((CHUNK * FIELDS,), jnp.float32),
            pltpu.VMEM((CHUNK,), jnp.float32),
            pltpu.VMEM((16,), jnp.float32),
            pltpu.SemaphoreType.DMA,
            pltpu.SemaphoreType.DMA,
        ],
        compiler_params=pltpu.CompilerParams(
            needs_layout_passes=False, use_tc_tiling_on_sc=False),
    )(xflat, table.reshape(N_VOCAB, K), bias.T.reshape(N_VOCAB), w0b)


def kernel(X, embeddings, bias, w0):
    return _fm_call(X.astype(jnp.int32), embeddings,
                    bias.astype(jnp.float32), w0.astype(jnp.float32))


# speedup vs baseline: 2.9027x; 1.0383x over previous
import jax
import jax.numpy as jnp
from jax import lax
from jax.experimental import pallas as pl
from jax.experimental.pallas import tpu as pltpu
from jax.experimental.pallas import tpu_sc as plsc

N_VOCAB = 1000000
K = 16
BATCH = 16384
FIELDS = 26

NC = 2
NS = 16
NW = NC * NS
SAMPLES_PER_W = BATCH // NW
CHUNK = 128
N_CHUNKS = SAMPLES_PER_W // CHUNK
IDX_ROWS = CHUNK * FIELDS // 128

TCOLS = 512
N_TCHUNKS = N_VOCAB // TCOLS
TAIL = N_VOCAB - N_TCHUNKS * TCOLS


def _transpose_body(embt_hbm, tail_hbm, out_hbm,
                    in0, in1, ou0, ou1, is0, is1, os0, os1):
    wid = lax.axis_index("s") * NC + lax.axis_index("c")
    iota16k = lax.iota(jnp.int32, 16) * K
    ins, outs, iss, oss = (in0, in1), (ou0, ou1), (is0, is1), (os0, os1)

    n_mine = (N_TCHUNKS - wid + NW - 1) // NW

    def col_off(j):
        return pl.multiple_of((wid + j * NW) * TCOLS, TCOLS)

    for b in range(2):
        pltpu.async_copy(embt_hbm.at[:, pl.ds(col_off(b), TCOLS)],
                         ins[b], iss[b])

    def half_body(i, _):
        for b in range(2):
            j = 2 * i + b

            @pl.when(j < n_mine)
            def _(b=b, j=j):
                co = col_off(j)
                pltpu.make_async_copy(
                    embt_hbm.at[:, pl.ds(co, TCOLS)], ins[b], iss[b]).wait()

                @pl.when(j >= 2)
                def _():
                    pltpu.make_async_copy(
                        outs[b], out_hbm.at[pl.ds(0, TCOLS * K)],
                        oss[b]).wait()

                def grp(g, _):
                    c = g * 16
                    cbase = c * K
                    for k in range(16):
                        vec = ins[b][k, pl.ds(c, 16)]
                        plsc.store_scatter(outs[b], [iota16k + (cbase + k)],
                                           vec)
                    return 0

                lax.fori_loop(0, TCOLS // 16, grp, 0)
                pltpu.async_copy(outs[b],
                                 out_hbm.at[pl.ds(co * K, TCOLS * K)], oss[b])

                @pl.when(j + 2 < n_mine)
                def _():
                    co2 = col_off(j + 2)
                    pltpu.async_copy(embt_hbm.at[:, pl.ds(co2, TCOLS)],
                                     ins[b], iss[b])
        return 0

    lax.fori_loop(0, (N_TCHUNKS // NW + 2) // 2, half_body, 0)

    for b in range(2):
        pltpu.make_async_copy(
            outs[b], out_hbm.at[pl.ds(0, TCOLS * K)], oss[b]).wait()

    @pl.when(wid == NW - 1)
    def _():
        pltpu.sync_copy(tail_hbm, out_hbm.at[pl.ds(N_TCHUNKS * TCOLS * K,
                                                   TAIL * K)])


def _fm_body(x_hbm, emb_hbm, bias_hbm, w0_hbm, out_hbm,
             idx0, idx1, emb0, emb1, bia0, bia1, ou0, ou1, w0_v,
             es0, es1, bs0, bs1, os0, os1):
    wid = lax.axis_index("s") * NC + lax.axis_index("c")

    pltpu.sync_copy(w0_hbm, w0_v)
    w0vec = w0_v[...]

    idxs, embs, bias_b = (idx0, idx1), (emb0, emb1), (bia0, bia1)
    outs, ess, bss, oss = (ou0, ou1), (es0, es1), (bs0, bs1), (os0, os1)

    iota = lax.iota(jnp.int32, 16)
    iota26 = iota * FIELDS
    zerof = jnp.zeros((16,), jnp.float32)
    CF = CHUNK * FIELDS

    def fire(c, b):
        off = (wid * N_CHUNKS + c) * CF
        pltpu.sync_copy(x_hbm.at[pl.ds(off, CF)], idxs[b])
        pltpu.async_copy(emb_hbm.at[idxs[b]], embs[b], ess[b])
        pltpu.async_copy(bias_hbm.at[idxs[b]], bias_b[b], bss[b])

    fire(0, 0)
    for c in range(N_CHUNKS):
        b = c % 2
        if c + 1 < N_CHUNKS:
            fire(c + 1, 1 - b)
        pltpu.make_async_copy(emb_hbm.at[idxs[b]], embs[b], ess[b]).wait()
        pltpu.make_async_copy(bias_hbm.at[idxs[b]], bias_b[b], bss[b]).wait()
        if c >= 2:
            pltpu.make_async_copy(
                outs[b], out_hbm.at[pl.ds(0, CHUNK)], oss[b]).wait()

        def group_body(g, _, b=b):
            rbase = iota26 + g * (16 * FIELDS)
            rows = [rbase + f for f in range(FIELDS)]

            def k_body(k, acc):
                cols = jnp.full((16,), k, jnp.int32)
                s = [zerof, zerof, zerof, zerof]
                q = [zerof, zerof, zerof, zerof]
                for f in range(FIELDS):
                    v = plsc.load_gather(embs[b], [rows[f], cols])
                    s[f % 4] = s[f % 4] + v
                    q[f % 4] = q[f % 4] + v * v
                st = (s[0] + s[1]) + (s[2] + s[3])
                qt = (q[0] + q[1]) + (q[2] + q[3])
                return acc + (st * st - qt)

            pair = lax.fori_loop(0, K, k_body, zerof)
            ba = [zerof, zerof]
            for f in range(FIELDS):
                ba[f % 2] = ba[f % 2] + plsc.load_gather(bias_b[b], [rows[f]])
            t = w0vec + (ba[0] + ba[1]) + 0.5 * pair
            outs[b][pl.ds(g * 16, 16)] = 5.5 / (1.0 + jnp.exp(-t))
            return 0

        lax.fori_loop(0, CHUNK // 16, group_body, 0)

        out_off = wid * SAMPLES_PER_W + c * CHUNK
        pltpu.async_copy(outs[b], out_hbm.at[pl.ds(out_off, CHUNK)], oss[b])

    for b in range(2):
        pltpu.make_async_copy(
            outs[b], out_hbm.at[pl.ds(0, CHUNK)], oss[b]).wait()


def _sc_mesh():
    return plsc.VectorSubcoreMesh(core_axis_name="c", subcore_axis_name="s")


@jax.jit
def _fm_call(X, emb, bias, w0):
    xflat = X.reshape(BATCH * FIELDS)
    w0b = jnp.broadcast_to(w0, (16,))
    tail = emb[N_TCHUNKS * TCOLS:, :].reshape(TAIL * K)

    table = pl.kernel(
        _transpose_body,
        out_type=jax.ShapeDtypeStruct((N_VOCAB * K,), jnp.float32),
        mesh=_sc_mesh(),
        scratch_types=[
            pltpu.VMEM((16, TCOLS), jnp.float32),
            pltpu.VMEM((16, TCOLS), jnp.float32),
            pltpu.VMEM((TCOLS * K,), jnp.float32),
            pltpu.VMEM((TCOLS * K,), jnp.float32),
            pltpu.SemaphoreType.DMA,
            pltpu.SemaphoreType.DMA,
            pltpu.SemaphoreType.DMA,
            pltpu.SemaphoreType.DMA,
        ],
        compiler_params=pltpu.CompilerParams(
            needs_layout_passes=False, use_tc_tiling_on_sc=True),
    )(emb.T, tail)

    return pl.kernel(
        _fm_body,
        out_type=jax.ShapeDtypeStruct((BATCH,), jnp.float32),
        mesh=_sc_mesh(),
        scratch_types=[
            pltpu.VMEM((CHUNK * FIELDS,), jnp.int32),
            pltpu.VMEM((CHUNK * FIELDS,), jnp.int32),
            pltpu.VMEM((CHUNK * FIELDS, K), jnp.float32),
            pltpu.VMEM((CHUNK * FIELDS, K), jnp.float32),
            pltpu.VMEM((CHUNK * FIELDS,), jnp.float32),
            pltpu.VMEM((CHUNK * FIELDS,), jnp.float32),
            pltpu.VMEM((CHUNK,), jnp.float32),
            pltpu.VMEM((CHUNK,), jnp.float32),
            pltpu.VMEM((16,), jnp.float32),
            pltpu.SemaphoreType.DMA,
            pltpu.SemaphoreType.DMA,
            pltpu.SemaphoreType.DMA,
            pltpu.SemaphoreType.DMA,
            pltpu.SemaphoreType.DMA,
            pltpu.SemaphoreType.DMA,
        ],
        compiler_params=pltpu.CompilerParams(
            needs_layout_passes=False, use_tc_tiling_on_sc=False),
    )(xflat, table.reshape(N_VOCAB, K), bias.T.reshape(N_VOCAB), w0b)


def kernel(X, embeddings, bias, w0):
    return _fm_call(X.astype(jnp.int32), embeddings,
                    bias.astype(jnp.float32), w0.astype(jnp.float32))
